# baseline (speedup 1.0000x reference)
import functools

import jax
import jax.numpy as jnp
from jax import lax
from jax.experimental import pallas as pl
from jax.experimental.pallas import tpu as pltpu
from jax.experimental.pallas import tpu_sc as plsc


@functools.lru_cache(maxsize=None)
def _make_gather(V, D, T, Bt):
    info = plsc.get_sparse_core_info()
    NC, NS = info.num_cores, info.num_subcores
    NW = NC * NS
    C = 1024
    while Bt % C != 0:
        C //= 2
    QP = Bt // C
    n_units = T * QP
    assert n_units % NW == 0
    per_w = n_units // NW
    mesh = plsc.VectorSubcoreMesh(core_axis_name="c", subcore_axis_name="s")

    @functools.partial(
        pl.kernel,
        mesh=mesh,
        out_type=jax.ShapeDtypeStruct((T, Bt, D), jnp.float32),
        scratch_types=[
            pltpu.VMEM((1, C), jnp.int32),
            pltpu.VMEM((1, C), jnp.int32),
            pltpu.VMEM((1, C, D), jnp.float32),
            pltpu.VMEM((1, C, D), jnp.float32),
            pltpu.SemaphoreType.DMA,
            pltpu.SemaphoreType.DMA,
            pltpu.SemaphoreType.DMA,
            pltpu.SemaphoreType.DMA,
            pltpu.SemaphoreType.DMA,
            pltpu.SemaphoreType.DMA,
        ],
        compiler_params=pltpu.CompilerParams(use_tc_tiling_on_sc=False),
    )
    def gather(
        table_hbm, idx_hbm, out_hbm,
        idx_v0, idx_v1, rows_v0, rows_v1,
        isem0, isem1, gsem0, gsem1, wsem0, wsem1,
    ):
        idx_v = (idx_v0, idx_v1)
        rows_v = (rows_v0, rows_v1)
        isem = (isem0, isem1)
        gsem = (gsem0, gsem1)
        wsem = (wsem0, wsem1)
        wid = lax.axis_index("s") * NC + lax.axis_index("c")

        def unit(k):
            u = wid + k * NW
            t = u // QP
            q = lax.rem(u, QP)
            return t, q * C

        def idx_slice(k):
            t, c0 = unit(k)
            return idx_hbm.at[pl.ds(t, 1), pl.ds(c0, C)]

        def out_slice(k):
            t, c0 = unit(k)
            return out_hbm.at[pl.ds(t, 1), pl.ds(c0, C), :]

        def start_idx(k):
            pltpu.async_copy(idx_slice(k), idx_v[k % 2], isem[k % 2])

        def wait_idx(k):
            pltpu.make_async_copy(idx_slice(k), idx_v[k % 2], isem[k % 2]).wait()

        def start_gather(k):
            s = k % 2
            pltpu.async_copy(table_hbm.at[idx_v[s].at[0]], rows_v[s].at[0], gsem[s])

        def wait_gather(k):
            s = k % 2
            pltpu.make_async_copy(
                table_hbm.at[idx_v[s].at[0]], rows_v[s].at[0], gsem[s]
            ).wait()

        def start_wb(k):
            pltpu.async_copy(rows_v[k % 2], out_slice(k), wsem[k % 2])

        def wait_wb(k):
            pltpu.make_async_copy(rows_v[k % 2], out_slice(k), wsem[k % 2]).wait()

        start_idx(0)
        if per_w > 1:
            start_idx(1)
        wait_idx(0)
        start_gather(0)
        for k in range(per_w):
            if k + 1 < per_w:
                wait_idx(k + 1)
                if k + 1 >= 2:
                    wait_wb(k - 1)
                start_gather(k + 1)
            wait_gather(k)
            if k + 2 < per_w:
                start_idx(k + 2)
            start_wb(k)
        for k in (per_w - 2, per_w - 1):
            if k >= 0:
                wait_wb(k)

    return gather


def kernel(input, table):
    T, Bt = input.shape
    V, D = table.shape
    return _make_gather(V, D, T, Bt)(table, input)
